# baseline (speedup 1.0000x reference)
import functools

import jax
import jax.numpy as jnp
from jax import lax
from jax.experimental import pallas as pl
from jax.experimental.pallas import tpu as pltpu
from jax.experimental.pallas import tpu_sc as plsc

_NUM_CORES = 2
_NUM_SUBCORES = 16
_NUM_WORKERS = _NUM_CORES * _NUM_SUBCORES


@functools.partial(jax.jit, static_argnames=("chunk", "nbuf"))
def _sc_embedding_lookup(ids, table, chunk=8, nbuf=4):
    bs, s = ids.shape
    _, d = table.shape
    n = bs * s
    per_w = n // _NUM_WORKERS
    w_per_row = s // per_w
    n_chunks = per_w // chunk
    n_steps = n_chunks // nbuf

    mesh = plsc.VectorSubcoreMesh(core_axis_name="c", subcore_axis_name="s")

    @functools.partial(
        pl.kernel,
        out_type=jax.ShapeDtypeStruct((bs, s, d), jnp.float32),
        mesh=mesh,
        scratch_types=[
            pltpu.VMEM((per_w,), jnp.int32),
            pltpu.VMEM((nbuf, chunk, d), jnp.float32),
            pltpu.SemaphoreType.DMA((nbuf,)),
            pltpu.SemaphoreType.DMA((nbuf,)),
        ],
    )
    def body(ids_hbm, table_hbm, out_hbm, idx_v, rows_v, gsem, osem):
        wid = lax.axis_index("s") * _NUM_CORES + lax.axis_index("c")
        row = wid // w_per_row
        col = (wid % w_per_row) * per_w
        pltpu.sync_copy(ids_hbm.at[row, pl.ds(col, per_w)], idx_v)

        def gather(c, b):
            off = pl.multiple_of(c * chunk, 8)
            return pltpu.make_async_copy(
                table_hbm.at[idx_v.at[pl.ds(off, chunk)]],
                rows_v.at[b],
                gsem.at[b],
            )

        def writeout(c, b):
            off = pl.multiple_of(c * chunk, 8)
            return pltpu.make_async_copy(
                rows_v.at[b],
                out_hbm.at[row, pl.ds(col + off, chunk)],
                osem.at[b],
            )

        def step(t, carry):
            for b in range(nbuf):
                c = t * nbuf + b

                @pl.when(t > 0)
                def _():
                    writeout(c - nbuf, b).wait()

                gather(c, b).start()
            for b in range(nbuf):
                c = t * nbuf + b
                gather(c, b).wait()
                writeout(c, b).start()
            return carry

        lax.fori_loop(0, n_steps, step, 0)
        rem = n_chunks - n_steps * nbuf
        for b in range(rem):
            c = n_steps * nbuf + b
            writeout(c - nbuf, b).wait()
            gather(c, b).start()
        for b in range(rem):
            c = n_steps * nbuf + b
            gather(c, b).wait()
            writeout(c, b).start()
            writeout(c, b).wait()
        for b in range(rem, nbuf):
            writeout((n_steps - 1) * nbuf + b, b).wait()

    return body(ids, table)


def kernel(input_ids, embed_tokens):
    return _sc_embedding_lookup(input_ids.astype(jnp.int32), embed_tokens)


# speedup vs baseline: 1.0114x; 1.0114x over previous
import functools

import jax
import jax.numpy as jnp
from jax import lax
from jax.experimental import pallas as pl
from jax.experimental.pallas import tpu as pltpu
from jax.experimental.pallas import tpu_sc as plsc

_NUM_CORES = 2
_NUM_SUBCORES = 16
_NUM_WORKERS = _NUM_CORES * _NUM_SUBCORES


@functools.partial(jax.jit, static_argnames=("chunk", "nbuf"))
def _sc_embedding_lookup(ids, table, chunk=8, nbuf=6):
    bs, s = ids.shape
    _, d = table.shape
    n = bs * s
    per_w = n // _NUM_WORKERS
    w_per_row = s // per_w
    n_chunks = per_w // chunk
    n_steps = n_chunks // nbuf

    mesh = plsc.VectorSubcoreMesh(core_axis_name="c", subcore_axis_name="s")

    @functools.partial(
        pl.kernel,
        out_type=jax.ShapeDtypeStruct((bs, s, d), jnp.float32),
        mesh=mesh,
        scratch_types=[
            pltpu.VMEM((per_w,), jnp.int32),
            pltpu.VMEM((nbuf, chunk, d), jnp.float32),
            pltpu.SemaphoreType.DMA((nbuf,)),
            pltpu.SemaphoreType.DMA((nbuf,)),
        ],
    )
    def body(ids_hbm, table_hbm, out_hbm, idx_v, rows_v, gsem, osem):
        wid = lax.axis_index("s") * _NUM_CORES + lax.axis_index("c")
        row = wid // w_per_row
        col = (wid % w_per_row) * per_w
        pltpu.sync_copy(ids_hbm.at[row, pl.ds(col, per_w)], idx_v)

        def gather(c, b):
            off = pl.multiple_of(c * chunk, 8)
            return pltpu.make_async_copy(
                table_hbm.at[idx_v.at[pl.ds(off, chunk)]],
                rows_v.at[b],
                gsem.at[b],
            )

        def writeout(c, b):
            off = pl.multiple_of(c * chunk, 8)
            return pltpu.make_async_copy(
                rows_v.at[b],
                out_hbm.at[row, pl.ds(col + off, chunk)],
                osem.at[b],
            )

        def step(t, carry):
            for b in range(nbuf):
                c = t * nbuf + b

                @pl.when(t > 0)
                def _():
                    writeout(c - nbuf, b).wait()

                gather(c, b).start()
            for b in range(nbuf):
                c = t * nbuf + b
                gather(c, b).wait()
                writeout(c, b).start()
            return carry

        lax.fori_loop(0, n_steps, step, 0)
        rem = n_chunks - n_steps * nbuf
        for b in range(rem):
            c = n_steps * nbuf + b
            writeout(c - nbuf, b).wait()
            gather(c, b).start()
        for b in range(rem):
            c = n_steps * nbuf + b
            gather(c, b).wait()
            writeout(c, b).start()
            writeout(c, b).wait()
        for b in range(rem, nbuf):
            writeout((n_steps - 1) * nbuf + b, b).wait()

    return body(ids, table)


def kernel(input_ids, embed_tokens):
    return _sc_embedding_lookup(input_ids.astype(jnp.int32), embed_tokens)
